# baseline (speedup 1.0000x reference)
import jax
import jax.numpy as jnp
from jax import lax
from jax.experimental import pallas as pl
from jax.experimental.pallas import tpu as pltpu
from jax.experimental.pallas import tpu_sc as plsc

N, H, W, C = 4, 384, 384, 96
NP = N * H * W
HW = H * W
NWORK = 32
PIX_PER_W = NP // NWORK
CHUNK = 64
NCH = PIX_PER_W // CHUNK
NSLOT = 3
LANES = 16
NVEC = C // LANES
CPAD = 128


def _warp_body(img_hbm, flow_hbm, out_hbm,
               f2v, iv, wv, bufs, acc, gsem, osem, fsem):
    c = lax.axis_index("c")
    s = lax.axis_index("s")
    wid = s * 2 + c
    base_px = wid * PIX_PER_W
    lanes = lax.iota(jnp.int32, LANES)

    def flow_fire(t):
        slot = t % NSLOT
        pb = base_px + t * CHUNK
        pltpu.async_copy(flow_hbm.at[pl.ds(pb, CHUNK), :],
                         f2v.at[slot], fsem.at[slot])

    def flow_wait(t):
        slot = t % NSLOT
        pltpu.make_async_copy(flow_hbm.at[pl.ds(0, CHUNK), :],
                              f2v.at[slot], fsem.at[slot]).wait()

    def idx_compute(t):
        slot = t % NSLOT
        pb = base_px + t * CHUNK
        g = pb // W
        n = g // H
        h = g % H
        col0 = pb % W
        nbase = n * HW
        hf = lax.convert_element_type(h, jnp.float32)
        for j in range(CHUNK // LANES):
            sl = pl.ds(j * LANES, LANES)
            px = j * LANES + lanes
            slotv = jnp.full((LANES,), slot, jnp.int32)
            fy = plsc.load_gather(f2v, [slotv, px,
                                        jnp.zeros((LANES,), jnp.int32)])
            fx = plsc.load_gather(f2v, [slotv, px,
                                        jnp.ones((LANES,), jnp.int32)])
            wcol = lax.convert_element_type(col0 + px, jnp.float32)
            qy = hf - fy
            qx = wcol - fx
            y0 = lax.convert_element_type(jnp.clip(qy, 0.0, float(H - 2)),
                                          jnp.int32)
            x0 = lax.convert_element_type(jnp.clip(qx, 0.0, float(W - 2)),
                                          jnp.int32)
            ay = jnp.clip(qy - lax.convert_element_type(y0, jnp.float32),
                          0.0, 1.0)
            ax = jnp.clip(qx - lax.convert_element_type(x0, jnp.float32),
                          0.0, 1.0)
            base = nbase + y0 * W + x0
            iv[slot, 0, sl] = base
            iv[slot, 1, sl] = base + 1
            iv[slot, 2, sl] = base + W
            iv[slot, 3, sl] = base + W + 1
            by = 1.0 - ay
            bx = 1.0 - ax
            wv[slot, 0, sl] = by * bx
            wv[slot, 1, sl] = by * ax
            wv[slot, 2, sl] = ay * bx
            wv[slot, 3, sl] = ay * ax

    def gather_fire(t):
        slot = t % NSLOT
        for q in range(4):
            pltpu.async_copy(img_hbm.at[iv.at[slot, q]], bufs.at[slot, q],
                             gsem.at[slot])

    def gather_wait(t):
        slot = t % NSLOT
        for q in range(4):
            pltpu.make_async_copy(img_hbm.at[iv.at[slot, q]],
                                  bufs.at[slot, q], gsem.at[slot]).wait()

    def blend(t):
        slot = t % NSLOT

        def grp_body(pg, carry2):
            pbase = pg * LANES
            v00 = wv[slot, 0, pl.ds(pbase, LANES)]
            v01 = wv[slot, 1, pl.ds(pbase, LANES)]
            v10 = wv[slot, 2, pl.ds(pbase, LANES)]
            v11 = wv[slot, 3, pl.ds(pbase, LANES)]
            for l in range(LANES):
                a00 = v00[l]
                a01 = v01[l]
                a10 = v10[l]
                a11 = v11[l]
                p = pbase + l
                for v in range(NVEC):
                    cs = pl.ds(v * LANES, LANES)
                    acc[slot, p, cs] = (
                        a00 * bufs[slot, 0, p, cs] + a01 * bufs[slot, 1, p, cs]
                        + a10 * bufs[slot, 2, p, cs]
                        + a11 * bufs[slot, 3, p, cs])
            return carry2

        lax.fori_loop(0, CHUNK // LANES, grp_body, 0)

    def out_fire(t):
        slot = t % NSLOT
        pb = base_px + t * CHUNK
        pltpu.async_copy(acc.at[slot], out_hbm.at[pl.ds(pb, CHUNK), :],
                         osem.at[slot])

    def out_wait(t):
        slot = t % NSLOT
        pltpu.make_async_copy(acc.at[slot],
                              out_hbm.at[pl.ds(0, CHUNK), :],
                              osem.at[slot]).wait()

    flow_fire(0)
    flow_fire(1)

    def step(u, carry):
        @pl.when(u < NCH)
        def _():
            flow_wait(u)
            idx_compute(u)
            gather_fire(u)

            @pl.when(u + 2 < NCH)
            def _():
                flow_fire(u + 2)

        @pl.when(u >= 1)
        def _():
            t = u - 1
            gather_wait(t)

            @pl.when(t >= NSLOT)
            def _():
                out_wait(t - NSLOT)

            blend(t)
            out_fire(t)

        return carry

    lax.fori_loop(0, NCH + 1, step, 0)
    for k in range(NSLOT):
        out_wait(NCH - NSLOT + k)


@jax.jit
def kernel(image, flow):
    img_flat = image.reshape(NP, C)
    flow_flat = flow.reshape(NP, 2)
    mesh = plsc.VectorSubcoreMesh(core_axis_name="c", subcore_axis_name="s")
    run = pl.kernel(
        _warp_body,
        out_type=jax.ShapeDtypeStruct((NP, CPAD), jnp.float32),
        mesh=mesh,
        compiler_params=pltpu.CompilerParams(use_tc_tiling_on_sc=False,
                                             needs_layout_passes=False),
        scratch_types=[
            pltpu.VMEM((NSLOT, CHUNK, 2), jnp.float32),
            pltpu.VMEM((NSLOT, 4, CHUNK), jnp.int32),
            pltpu.VMEM((NSLOT, 4, CHUNK), jnp.float32),
            pltpu.VMEM((NSLOT, 4, CHUNK, C), jnp.float32),
            pltpu.VMEM((NSLOT, CHUNK, CPAD), jnp.float32),
            pltpu.SemaphoreType.DMA((NSLOT,)),
            pltpu.SemaphoreType.DMA((NSLOT,)),
            pltpu.SemaphoreType.DMA((NSLOT,)),
        ],
    )
    return run(img_flat, flow_flat)[:, :C].reshape(N, H, W, C)


# speedup vs baseline: 1.2833x; 1.2833x over previous
import jax
import jax.numpy as jnp
from jax import lax
from jax.experimental import pallas as pl
from jax.experimental.pallas import tpu as pltpu
from jax.experimental.pallas import tpu_sc as plsc

N, H, W, C = 4, 384, 384, 96
NP = N * H * W
HW = H * W
NWORK = 32
PIX_PER_W = NP // NWORK
CHUNK = 64
NCH = PIX_PER_W // CHUNK
NSLOT = 3
LANES = 16
NVEC = C // LANES
CPAD = 128


def _warp_body(img_hbm, fyfx_hbm, out_hbm,
               fyv, fxv, iv, wv, bufs, acc, gsem, osem, fsem):
    c = lax.axis_index("c")
    s = lax.axis_index("s")
    wid = s * 2 + c
    base_px = wid * PIX_PER_W
    lanes = lax.iota(jnp.int32, LANES)

    def flow_fire(t):
        slot = t % NSLOT
        pb = base_px + t * CHUNK
        pltpu.async_copy(fyfx_hbm.at[0, pl.ds(pb, CHUNK)], fyv.at[slot],
                         fsem.at[slot])
        pltpu.async_copy(fyfx_hbm.at[1, pl.ds(pb, CHUNK)], fxv.at[slot],
                         fsem.at[slot])

    def flow_wait(t):
        slot = t % NSLOT
        pltpu.make_async_copy(fyfx_hbm.at[0, pl.ds(0, CHUNK)], fyv.at[slot],
                              fsem.at[slot]).wait()
        pltpu.make_async_copy(fyfx_hbm.at[1, pl.ds(0, CHUNK)], fxv.at[slot],
                              fsem.at[slot]).wait()

    def idx_compute(t):
        slot = t % NSLOT
        pb = base_px + t * CHUNK
        g = pb // W
        n = g // H
        h = g % H
        col0 = pb % W
        nbase = n * HW
        hf = lax.convert_element_type(h, jnp.float32)
        for j in range(CHUNK // LANES):
            sl = pl.ds(j * LANES, LANES)
            fy = fyv[slot, sl]
            fx = fxv[slot, sl]
            px = j * LANES + lanes
            wcol = lax.convert_element_type(col0 + px, jnp.float32)
            qy = hf - fy
            qx = wcol - fx
            y0 = lax.convert_element_type(jnp.clip(qy, 0.0, float(H - 2)),
                                          jnp.int32)
            x0 = lax.convert_element_type(jnp.clip(qx, 0.0, float(W - 2)),
                                          jnp.int32)
            ay = jnp.clip(qy - lax.convert_element_type(y0, jnp.float32),
                          0.0, 1.0)
            ax = jnp.clip(qx - lax.convert_element_type(x0, jnp.float32),
                          0.0, 1.0)
            base = nbase + y0 * W + x0
            iv[slot, 0, sl] = base
            iv[slot, 1, sl] = base + 1
            iv[slot, 2, sl] = base + W
            iv[slot, 3, sl] = base + W + 1
            by = 1.0 - ay
            bx = 1.0 - ax
            wv[slot, 0, sl] = by * bx
            wv[slot, 1, sl] = by * ax
            wv[slot, 2, sl] = ay * bx
            wv[slot, 3, sl] = ay * ax

    def gather_fire(t):
        slot = t % NSLOT
        for q in range(4):
            pltpu.async_copy(img_hbm.at[iv.at[slot, q]], bufs.at[slot, q],
                             gsem.at[slot])

    def gather_wait(t):
        slot = t % NSLOT
        for q in range(4):
            pltpu.make_async_copy(img_hbm.at[iv.at[slot, q]],
                                  bufs.at[slot, q], gsem.at[slot]).wait()

    def blend(t):
        slot = t % NSLOT

        def grp_body(pg, carry2):
            pbase = pg * LANES
            v00 = wv[slot, 0, pl.ds(pbase, LANES)]
            v01 = wv[slot, 1, pl.ds(pbase, LANES)]
            v10 = wv[slot, 2, pl.ds(pbase, LANES)]
            v11 = wv[slot, 3, pl.ds(pbase, LANES)]
            for l in range(LANES):
                a00 = v00[l]
                a01 = v01[l]
                a10 = v10[l]
                a11 = v11[l]
                p = pbase + l
                for v in range(NVEC):
                    cs = pl.ds(v * LANES, LANES)
                    acc[slot, p, cs] = (
                        a00 * bufs[slot, 0, p, cs] + a01 * bufs[slot, 1, p, cs]
                        + a10 * bufs[slot, 2, p, cs]
                        + a11 * bufs[slot, 3, p, cs])
            return carry2

        lax.fori_loop(0, CHUNK // LANES, grp_body, 0)

    def out_fire(t):
        slot = t % NSLOT
        pb = base_px + t * CHUNK
        pltpu.async_copy(acc.at[slot], out_hbm.at[pl.ds(pb, CHUNK), :],
                         osem.at[slot])

    def out_wait(t):
        slot = t % NSLOT
        pltpu.make_async_copy(acc.at[slot],
                              out_hbm.at[pl.ds(0, CHUNK), :],
                              osem.at[slot]).wait()

    flow_fire(0)
    flow_fire(1)

    def step(u, carry):
        @pl.when(u < NCH)
        def _():
            flow_wait(u)
            idx_compute(u)
            gather_fire(u)

            @pl.when(u + 2 < NCH)
            def _():
                flow_fire(u + 2)

        @pl.when(u >= 1)
        def _():
            t = u - 1
            gather_wait(t)

            @pl.when(t >= NSLOT)
            def _():
                out_wait(t - NSLOT)

            blend(t)
            out_fire(t)

        return carry

    lax.fori_loop(0, NCH + 1, step, 0)
    for k in range(NSLOT):
        out_wait(NCH - NSLOT + k)


@jax.jit
def kernel(image, flow):
    img_flat = image.reshape(NP, C)
    fyfx = jnp.moveaxis(flow, 3, 0).reshape(2, NP)
    mesh = plsc.VectorSubcoreMesh(core_axis_name="c", subcore_axis_name="s")
    run = pl.kernel(
        _warp_body,
        out_type=jax.ShapeDtypeStruct((NP, CPAD), jnp.float32),
        mesh=mesh,
        compiler_params=pltpu.CompilerParams(use_tc_tiling_on_sc=False),
        scratch_types=[
            pltpu.VMEM((NSLOT, CHUNK), jnp.float32),
            pltpu.VMEM((NSLOT, CHUNK), jnp.float32),
            pltpu.VMEM((NSLOT, 4, CHUNK), jnp.int32),
            pltpu.VMEM((NSLOT, 4, CHUNK), jnp.float32),
            pltpu.VMEM((NSLOT, 4, CHUNK, C), jnp.float32),
            pltpu.VMEM((NSLOT, CHUNK, CPAD), jnp.float32),
            pltpu.SemaphoreType.DMA((NSLOT,)),
            pltpu.SemaphoreType.DMA((NSLOT,)),
            pltpu.SemaphoreType.DMA((NSLOT,)),
        ],
    )
    return run(img_flat, fyfx)[:, :C].reshape(N, H, W, C)


# speedup vs baseline: 1.2846x; 1.0011x over previous
import jax
import jax.numpy as jnp
from jax import lax
from jax.experimental import pallas as pl
from jax.experimental.pallas import tpu as pltpu
from jax.experimental.pallas import tpu_sc as plsc

N, H, W, C = 4, 384, 384, 96
NP = N * H * W
HW = H * W
NWORK = 32
PIX_PER_W = NP // NWORK
CHUNK = 64
NCH = PIX_PER_W // CHUNK
NSLOT = 3
LANES = 16
NVEC = C // LANES
CPAD = 128


def _warp_body(img_hbm, fyfx_hbm, out_hbm,
               fyv, fxv, iv, wv, bufs, acc, gsem, osem, fsem):
    c = lax.axis_index("c")
    s = lax.axis_index("s")
    wid = s * 2 + c
    base_px = wid * PIX_PER_W
    lanes = lax.iota(jnp.int32, LANES)

    def flow_fire(t):
        slot = t % NSLOT
        pb = base_px + t * CHUNK
        pltpu.async_copy(fyfx_hbm.at[0, pl.ds(pb, CHUNK)], fyv.at[slot],
                         fsem.at[slot])
        pltpu.async_copy(fyfx_hbm.at[1, pl.ds(pb, CHUNK)], fxv.at[slot],
                         fsem.at[slot])

    def flow_wait(t):
        slot = t % NSLOT
        pltpu.make_async_copy(fyfx_hbm.at[0, pl.ds(0, CHUNK)], fyv.at[slot],
                              fsem.at[slot]).wait()
        pltpu.make_async_copy(fyfx_hbm.at[1, pl.ds(0, CHUNK)], fxv.at[slot],
                              fsem.at[slot]).wait()

    def idx_compute(t):
        slot = t % NSLOT
        pb = base_px + t * CHUNK
        g = pb // W
        n = g // H
        h = g % H
        col0 = pb % W
        nbase = n * HW
        hf = lax.convert_element_type(h, jnp.float32)
        for j in range(CHUNK // LANES):
            sl = pl.ds(j * LANES, LANES)
            fy = fyv[slot, sl]
            fx = fxv[slot, sl]
            px = j * LANES + lanes
            wcol = lax.convert_element_type(col0 + px, jnp.float32)
            qy = hf - fy
            qx = wcol - fx
            y0 = lax.convert_element_type(jnp.clip(qy, 0.0, float(H - 2)),
                                          jnp.int32)
            x0 = lax.convert_element_type(jnp.clip(qx, 0.0, float(W - 2)),
                                          jnp.int32)
            ay = jnp.clip(qy - lax.convert_element_type(y0, jnp.float32),
                          0.0, 1.0)
            ax = jnp.clip(qx - lax.convert_element_type(x0, jnp.float32),
                          0.0, 1.0)
            base = nbase + y0 * W + x0
            iv[slot, 0, sl] = base
            iv[slot, 1, sl] = base + 1
            iv[slot, 2, sl] = base + W
            iv[slot, 3, sl] = base + W + 1
            by = 1.0 - ay
            bx = 1.0 - ax
            wv[slot, 0, sl] = by * bx
            wv[slot, 1, sl] = by * ax
            wv[slot, 2, sl] = ay * bx
            wv[slot, 3, sl] = ay * ax

    def gather_fire(t):
        slot = t % NSLOT
        for q in range(4):
            pltpu.async_copy(img_hbm.at[iv.at[slot, q]], bufs.at[slot, q],
                             gsem.at[slot])

    def gather_wait(t):
        slot = t % NSLOT
        for q in range(4):
            pltpu.make_async_copy(img_hbm.at[iv.at[slot, q]],
                                  bufs.at[slot, q], gsem.at[slot]).wait()

    def blend(t):
        slot = t % NSLOT

        def grp_body(pg, carry2):
            pbase = pg * LANES
            v00 = wv[slot, 0, pl.ds(pbase, LANES)]
            v01 = wv[slot, 1, pl.ds(pbase, LANES)]
            v10 = wv[slot, 2, pl.ds(pbase, LANES)]
            v11 = wv[slot, 3, pl.ds(pbase, LANES)]
            for l in range(LANES):
                a00 = v00[l]
                a01 = v01[l]
                a10 = v10[l]
                a11 = v11[l]
                p = pbase + l
                for v in range(NVEC):
                    cs = pl.ds(v * LANES, LANES)
                    acc[slot, p, cs] = (
                        a00 * bufs[slot, 0, p, cs] + a01 * bufs[slot, 1, p, cs]
                        + a10 * bufs[slot, 2, p, cs]
                        + a11 * bufs[slot, 3, p, cs])
            return carry2

        lax.fori_loop(0, CHUNK // LANES, grp_body, 0)

    def out_fire(t):
        slot = t % NSLOT
        pb = base_px + t * CHUNK
        pltpu.async_copy(acc.at[slot], out_hbm.at[pl.ds(pb, CHUNK), pl.ds(0, C)],
                         osem.at[slot])

    def out_wait(t):
        slot = t % NSLOT
        pltpu.make_async_copy(acc.at[slot],
                              out_hbm.at[pl.ds(0, CHUNK), pl.ds(0, C)],
                              osem.at[slot]).wait()

    flow_fire(0)
    flow_fire(1)

    def step(u, carry):
        @pl.when(u < NCH)
        def _():
            flow_wait(u)
            idx_compute(u)
            gather_fire(u)

            @pl.when(u + 2 < NCH)
            def _():
                flow_fire(u + 2)

        @pl.when(u >= 1)
        def _():
            t = u - 1
            gather_wait(t)

            @pl.when(t >= NSLOT)
            def _():
                out_wait(t - NSLOT)

            blend(t)
            out_fire(t)

        return carry

    lax.fori_loop(0, NCH + 1, step, 0)
    for k in range(NSLOT):
        out_wait(NCH - NSLOT + k)


@jax.jit
def kernel(image, flow):
    img_flat = image.reshape(NP, C)
    fyfx = jnp.moveaxis(flow, 3, 0).reshape(2, NP)
    mesh = plsc.VectorSubcoreMesh(core_axis_name="c", subcore_axis_name="s")
    run = pl.kernel(
        _warp_body,
        out_type=jax.ShapeDtypeStruct((NP, CPAD), jnp.float32),
        mesh=mesh,
        compiler_params=pltpu.CompilerParams(use_tc_tiling_on_sc=False),
        scratch_types=[
            pltpu.VMEM((NSLOT, CHUNK), jnp.float32),
            pltpu.VMEM((NSLOT, CHUNK), jnp.float32),
            pltpu.VMEM((NSLOT, 4, CHUNK), jnp.int32),
            pltpu.VMEM((NSLOT, 4, CHUNK), jnp.float32),
            pltpu.VMEM((NSLOT, 4, CHUNK, C), jnp.float32),
            pltpu.VMEM((NSLOT, CHUNK, C), jnp.float32),
            pltpu.SemaphoreType.DMA((NSLOT,)),
            pltpu.SemaphoreType.DMA((NSLOT,)),
            pltpu.SemaphoreType.DMA((NSLOT,)),
        ],
    )
    return run(img_flat, fyfx)[:, :C].reshape(N, H, W, C)


# speedup vs baseline: 1.2855x; 1.0007x over previous
import jax
import jax.numpy as jnp
from jax import lax
from jax.experimental import pallas as pl
from jax.experimental.pallas import tpu as pltpu
from jax.experimental.pallas import tpu_sc as plsc

N, H, W, C = 4, 384, 384, 96
NP = N * H * W
HW = H * W
NWORK = 32
PIX_PER_W = NP // NWORK
CHUNK = 64
NCH = PIX_PER_W // CHUNK
NSLOT = 4
LANES = 16
NVEC = C // LANES
CPAD = 128


def _warp_body(img_hbm, fyfx_hbm, out_hbm,
               fyv, fxv, iv, wv, bufs, acc, gsem, osem, fsem):
    c = lax.axis_index("c")
    s = lax.axis_index("s")
    wid = s * 2 + c
    base_px = wid * PIX_PER_W
    lanes = lax.iota(jnp.int32, LANES)

    def flow_fire(t):
        slot = t % NSLOT
        pb = base_px + t * CHUNK
        pltpu.async_copy(fyfx_hbm.at[0, pl.ds(pb, CHUNK)], fyv.at[slot],
                         fsem.at[slot])
        pltpu.async_copy(fyfx_hbm.at[1, pl.ds(pb, CHUNK)], fxv.at[slot],
                         fsem.at[slot])

    def flow_wait(t):
        slot = t % NSLOT
        pltpu.make_async_copy(fyfx_hbm.at[0, pl.ds(0, CHUNK)], fyv.at[slot],
                              fsem.at[slot]).wait()
        pltpu.make_async_copy(fyfx_hbm.at[1, pl.ds(0, CHUNK)], fxv.at[slot],
                              fsem.at[slot]).wait()

    def idx_compute(t):
        slot = t % NSLOT
        pb = base_px + t * CHUNK
        g = pb // W
        n = g // H
        h = g % H
        col0 = pb % W
        nbase = n * HW
        hf = lax.convert_element_type(h, jnp.float32)
        for j in range(CHUNK // LANES):
            sl = pl.ds(j * LANES, LANES)
            fy = fyv[slot, sl]
            fx = fxv[slot, sl]
            px = j * LANES + lanes
            wcol = lax.convert_element_type(col0 + px, jnp.float32)
            qy = hf - fy
            qx = wcol - fx
            y0 = lax.convert_element_type(jnp.clip(qy, 0.0, float(H - 2)),
                                          jnp.int32)
            x0 = lax.convert_element_type(jnp.clip(qx, 0.0, float(W - 2)),
                                          jnp.int32)
            ay = jnp.clip(qy - lax.convert_element_type(y0, jnp.float32),
                          0.0, 1.0)
            ax = jnp.clip(qx - lax.convert_element_type(x0, jnp.float32),
                          0.0, 1.0)
            base = nbase + y0 * W + x0
            iv[slot, 0, sl] = base
            iv[slot, 1, sl] = base + 1
            iv[slot, 2, sl] = base + W
            iv[slot, 3, sl] = base + W + 1
            by = 1.0 - ay
            bx = 1.0 - ax
            wv[slot, 0, sl] = by * bx
            wv[slot, 1, sl] = by * ax
            wv[slot, 2, sl] = ay * bx
            wv[slot, 3, sl] = ay * ax

    def gather_fire(t):
        slot = t % NSLOT
        for q in range(4):
            pltpu.async_copy(img_hbm.at[iv.at[slot, q]], bufs.at[slot, q],
                             gsem.at[slot])

    def gather_wait(t):
        slot = t % NSLOT
        for q in range(4):
            pltpu.make_async_copy(img_hbm.at[iv.at[slot, q]],
                                  bufs.at[slot, q], gsem.at[slot]).wait()

    def blend(t):
        slot = t % NSLOT

        def grp_body(pg, carry2):
            pbase = pg * LANES
            v00 = wv[slot, 0, pl.ds(pbase, LANES)]
            v01 = wv[slot, 1, pl.ds(pbase, LANES)]
            v10 = wv[slot, 2, pl.ds(pbase, LANES)]
            v11 = wv[slot, 3, pl.ds(pbase, LANES)]
            for l in range(LANES):
                a00 = v00[l]
                a01 = v01[l]
                a10 = v10[l]
                a11 = v11[l]
                p = pbase + l
                for v in range(NVEC):
                    cs = pl.ds(v * LANES, LANES)
                    acc[slot, p, cs] = (
                        a00 * bufs[slot, 0, p, cs] + a01 * bufs[slot, 1, p, cs]
                        + a10 * bufs[slot, 2, p, cs]
                        + a11 * bufs[slot, 3, p, cs])
            return carry2

        lax.fori_loop(0, CHUNK // LANES, grp_body, 0)

    def out_fire(t):
        slot = t % NSLOT
        pb = base_px + t * CHUNK
        pltpu.async_copy(acc.at[slot], out_hbm.at[pl.ds(pb, CHUNK), pl.ds(0, C)],
                         osem.at[slot])

    def out_wait(t):
        slot = t % NSLOT
        pltpu.make_async_copy(acc.at[slot],
                              out_hbm.at[pl.ds(0, CHUNK), pl.ds(0, C)],
                              osem.at[slot]).wait()

    flow_fire(0)
    flow_fire(1)

    def step(u, carry):
        @pl.when(u < NCH)
        def _():
            flow_wait(u)
            idx_compute(u)
            gather_fire(u)

            @pl.when(u + 2 < NCH)
            def _():
                flow_fire(u + 2)

        @pl.when(u >= 1)
        def _():
            t = u - 1
            gather_wait(t)

            @pl.when(t >= NSLOT)
            def _():
                out_wait(t - NSLOT)

            blend(t)
            out_fire(t)

        return carry

    lax.fori_loop(0, NCH + 1, step, 0)
    for k in range(NSLOT):
        out_wait(NCH - NSLOT + k)


@jax.jit
def kernel(image, flow):
    img_flat = image.reshape(NP, C)
    fyfx = jnp.moveaxis(flow, 3, 0).reshape(2, NP)
    mesh = plsc.VectorSubcoreMesh(core_axis_name="c", subcore_axis_name="s")
    run = pl.kernel(
        _warp_body,
        out_type=jax.ShapeDtypeStruct((NP, CPAD), jnp.float32),
        mesh=mesh,
        compiler_params=pltpu.CompilerParams(use_tc_tiling_on_sc=False),
        scratch_types=[
            pltpu.VMEM((NSLOT, CHUNK), jnp.float32),
            pltpu.VMEM((NSLOT, CHUNK), jnp.float32),
            pltpu.VMEM((NSLOT, 4, CHUNK), jnp.int32),
            pltpu.VMEM((NSLOT, 4, CHUNK), jnp.float32),
            pltpu.VMEM((NSLOT, 4, CHUNK, C), jnp.float32),
            pltpu.VMEM((NSLOT, CHUNK, C), jnp.float32),
            pltpu.SemaphoreType.DMA((NSLOT,)),
            pltpu.SemaphoreType.DMA((NSLOT,)),
            pltpu.SemaphoreType.DMA((NSLOT,)),
        ],
    )
    return run(img_flat, fyfx)[:, :C].reshape(N, H, W, C)


# speedup vs baseline: 1.2878x; 1.0018x over previous
import jax
import jax.numpy as jnp
from jax import lax
from jax.experimental import pallas as pl
from jax.experimental.pallas import tpu as pltpu
from jax.experimental.pallas import tpu_sc as plsc

N, H, W, C = 4, 384, 384, 96
NP = N * H * W
HW = H * W
NWORK = 32
PIX_PER_W = NP // NWORK
CHUNK = 128
NCH = PIX_PER_W // CHUNK
NSLOT = 2
LANES = 16
NVEC = C // LANES
CPAD = 128


def _warp_body(img_hbm, fyfx_hbm, out_hbm,
               fyv, fxv, iv, wv, bufs, acc, gsem, osem, fsem):
    c = lax.axis_index("c")
    s = lax.axis_index("s")
    wid = s * 2 + c
    base_px = wid * PIX_PER_W
    lanes = lax.iota(jnp.int32, LANES)

    def flow_fire(t):
        slot = t % NSLOT
        pb = base_px + t * CHUNK
        pltpu.async_copy(fyfx_hbm.at[0, pl.ds(pb, CHUNK)], fyv.at[slot],
                         fsem.at[slot])
        pltpu.async_copy(fyfx_hbm.at[1, pl.ds(pb, CHUNK)], fxv.at[slot],
                         fsem.at[slot])

    def flow_wait(t):
        slot = t % NSLOT
        pltpu.make_async_copy(fyfx_hbm.at[0, pl.ds(0, CHUNK)], fyv.at[slot],
                              fsem.at[slot]).wait()
        pltpu.make_async_copy(fyfx_hbm.at[1, pl.ds(0, CHUNK)], fxv.at[slot],
                              fsem.at[slot]).wait()

    def idx_compute(t):
        slot = t % NSLOT
        pb = base_px + t * CHUNK
        g = pb // W
        n = g // H
        h = g % H
        col0 = pb % W
        nbase = n * HW
        hf = lax.convert_element_type(h, jnp.float32)
        for j in range(CHUNK // LANES):
            sl = pl.ds(j * LANES, LANES)
            fy = fyv[slot, sl]
            fx = fxv[slot, sl]
            px = j * LANES + lanes
            wcol = lax.convert_element_type(col0 + px, jnp.float32)
            qy = hf - fy
            qx = wcol - fx
            y0 = lax.convert_element_type(jnp.clip(qy, 0.0, float(H - 2)),
                                          jnp.int32)
            x0 = lax.convert_element_type(jnp.clip(qx, 0.0, float(W - 2)),
                                          jnp.int32)
            ay = jnp.clip(qy - lax.convert_element_type(y0, jnp.float32),
                          0.0, 1.0)
            ax = jnp.clip(qx - lax.convert_element_type(x0, jnp.float32),
                          0.0, 1.0)
            base = nbase + y0 * W + x0
            iv[slot, 0, sl] = base
            iv[slot, 1, sl] = base + 1
            iv[slot, 2, sl] = base + W
            iv[slot, 3, sl] = base + W + 1
            by = 1.0 - ay
            bx = 1.0 - ax
            wv[slot, 0, sl] = by * bx
            wv[slot, 1, sl] = by * ax
            wv[slot, 2, sl] = ay * bx
            wv[slot, 3, sl] = ay * ax

    def gather_fire(t):
        slot = t % NSLOT
        for q in range(4):
            pltpu.async_copy(img_hbm.at[iv.at[slot, q]], bufs.at[slot, q],
                             gsem.at[slot])

    def gather_wait(t):
        slot = t % NSLOT
        for q in range(4):
            pltpu.make_async_copy(img_hbm.at[iv.at[slot, q]],
                                  bufs.at[slot, q], gsem.at[slot]).wait()

    def blend(t):
        slot = t % NSLOT

        def grp_body(pg, carry2):
            pbase = pg * LANES
            v00 = wv[slot, 0, pl.ds(pbase, LANES)]
            v01 = wv[slot, 1, pl.ds(pbase, LANES)]
            v10 = wv[slot, 2, pl.ds(pbase, LANES)]
            v11 = wv[slot, 3, pl.ds(pbase, LANES)]
            for l in range(LANES):
                a00 = v00[l]
                a01 = v01[l]
                a10 = v10[l]
                a11 = v11[l]
                p = pbase + l
                for v in range(NVEC):
                    cs = pl.ds(v * LANES, LANES)
                    acc[slot, p, cs] = (
                        a00 * bufs[slot, 0, p, cs] + a01 * bufs[slot, 1, p, cs]
                        + a10 * bufs[slot, 2, p, cs]
                        + a11 * bufs[slot, 3, p, cs])
            return carry2

        lax.fori_loop(0, CHUNK // LANES, grp_body, 0)

    def out_fire(t):
        slot = t % NSLOT
        pb = base_px + t * CHUNK
        pltpu.async_copy(acc.at[slot], out_hbm.at[pl.ds(pb, CHUNK), pl.ds(0, C)],
                         osem.at[slot])

    def out_wait(t):
        slot = t % NSLOT
        pltpu.make_async_copy(acc.at[slot],
                              out_hbm.at[pl.ds(0, CHUNK), pl.ds(0, C)],
                              osem.at[slot]).wait()

    flow_fire(0)
    flow_fire(1)

    def step(u, carry):
        @pl.when(u < NCH)
        def _():
            flow_wait(u)
            idx_compute(u)
            gather_fire(u)

            @pl.when(u + 2 < NCH)
            def _():
                flow_fire(u + 2)

        @pl.when(u >= 1)
        def _():
            t = u - 1
            gather_wait(t)

            @pl.when(t >= NSLOT)
            def _():
                out_wait(t - NSLOT)

            blend(t)
            out_fire(t)

        return carry

    lax.fori_loop(0, NCH + 1, step, 0)
    for k in range(NSLOT):
        out_wait(NCH - NSLOT + k)


@jax.jit
def kernel(image, flow):
    img_flat = image.reshape(NP, C)
    fyfx = jnp.moveaxis(flow, 3, 0).reshape(2, NP)
    mesh = plsc.VectorSubcoreMesh(core_axis_name="c", subcore_axis_name="s")
    run = pl.kernel(
        _warp_body,
        out_type=jax.ShapeDtypeStruct((NP, CPAD), jnp.float32),
        mesh=mesh,
        compiler_params=pltpu.CompilerParams(use_tc_tiling_on_sc=False),
        scratch_types=[
            pltpu.VMEM((NSLOT, CHUNK), jnp.float32),
            pltpu.VMEM((NSLOT, CHUNK), jnp.float32),
            pltpu.VMEM((NSLOT, 4, CHUNK), jnp.int32),
            pltpu.VMEM((NSLOT, 4, CHUNK), jnp.float32),
            pltpu.VMEM((NSLOT, 4, CHUNK, C), jnp.float32),
            pltpu.VMEM((NSLOT, CHUNK, C), jnp.float32),
            pltpu.SemaphoreType.DMA((NSLOT,)),
            pltpu.SemaphoreType.DMA((NSLOT,)),
            pltpu.SemaphoreType.DMA((NSLOT,)),
        ],
    )
    return run(img_flat, fyfx)[:, :C].reshape(N, H, W, C)
